# Initial kernel scaffold; baseline (speedup 1.0000x reference)
import functools

import jax
import jax.numpy as jnp
from jax import lax
from jax.experimental import pallas as pl
from jax.experimental.pallas import tpu as pltpu
from jax.experimental.pallas import tpu_sc as plsc

N = 10000
E = 320000
D = 128
H1 = 32
H2 = 16
C = 7

NC = 2
NS = 16
NW = NC * NS
K = 128
WCHUNKS = 79
NCHUNKS = NW * WCHUNKS
EPAD = NCHUNKS * K
NPAD = 10008
NP1 = 10016

BN = 2000
GRID = N // BN
CP = 8


def _sc_mesh():
    return plsc.VectorSubcoreMesh(core_axis_name="c", subcore_axis_name="s")


@functools.partial(
    pl.kernel,
    out_type=jax.ShapeDtypeStruct((NW, NP1), jnp.float32),
    mesh=_sc_mesh(),
    scratch_types=[
        pltpu.VMEM((WCHUNKS, K), jnp.int32),
        pltpu.VMEM((NP1,), jnp.float32),
    ],
)
def _deg_kernel(dsts_hbm, zeros_hbm, out_hbm, dst_v, deg_v):
    c = lax.axis_index("c")
    s = lax.axis_index("s")
    wid = s * NC + c
    pltpu.sync_copy(zeros_hbm, deg_v)
    pltpu.sync_copy(dsts_hbm.at[pl.ds(wid * WCHUNKS, WCHUNKS)], dst_v)
    ones = jnp.full((16,), 1.0, dtype=jnp.float32)

    def body(j, carry):
        row = dst_v.at[j]
        for g in range(K // 16):
            idx = row[pl.ds(g * 16, 16)]
            plsc.addupdate_scatter(deg_v, [idx], ones)
        return carry

    lax.fori_loop(0, WCHUNKS, body, 0)
    pltpu.sync_copy(deg_v, out_hbm.at[wid])


def _make_agg(H):
    @functools.partial(
        pl.kernel,
        out_type=jax.ShapeDtypeStruct((NC, N, H), jnp.float32),
        mesh=_sc_mesh(),
        scratch_types=[
            pltpu.VMEM((WCHUNKS, K), jnp.int32),
            pltpu.VMEM((WCHUNKS, K), jnp.int32),
            pltpu.VMEM((K, H), jnp.float32),
            pltpu.VMEM_SHARED((NPAD, H), jnp.float32),
            pltpu.SemaphoreType.DMA,
        ],
    )
    def agg(hs_hbm, srcs_hbm, dsts_hbm, zeros_hbm, out_hbm,
            src_v, dst_v, rows_v, acc, sem):
        c = lax.axis_index("c")
        s = lax.axis_index("s")
        wid = s * NC + c

        @pl.when(s == 0)
        def _():
            pltpu.sync_copy(zeros_hbm, acc)

        pltpu.sync_copy(srcs_hbm.at[pl.ds(wid * WCHUNKS, WCHUNKS)], src_v)
        pltpu.sync_copy(dsts_hbm.at[pl.ds(wid * WCHUNKS, WCHUNKS)], dst_v)
        plsc.subcore_barrier()

        def body(j, carry):
            pltpu.async_copy(hs_hbm.at[src_v.at[j]], rows_v, sem).wait()
            pltpu.sync_copy(rows_v, acc.at[dst_v.at[j]], add=True)
            return carry

        lax.fori_loop(0, WCHUNKS, body, 0)
        plsc.subcore_barrier()

        @pl.when(s == 0)
        def _():
            pltpu.sync_copy(acc.at[pl.ds(0, N)], out_hbm.at[c])

    return agg


_agg_h1 = _make_agg(H1)
_agg_h2 = _make_agg(H2)


def _dense1_body(x_ref, w_ref, degp_ref, hs_ref, dinv_ref):
    p = degp_ref[...]
    ones = jnp.ones((NW, 1), dtype=jnp.float32)
    deg = lax.dot_general(p, ones, (((0,), (0,)), ((), ())),
                          preferred_element_type=jnp.float32)
    dinv = lax.rsqrt(deg + 1.0)
    h = jnp.dot(x_ref[...], w_ref[...], preferred_element_type=jnp.float32)
    hs_ref[...] = h * dinv
    dinv_ref[...] = dinv


def _dense1(x, W1, degp):
    return pl.pallas_call(
        _dense1_body,
        grid=(GRID,),
        in_specs=[
            pl.BlockSpec((BN, D), lambda i: (i, 0)),
            pl.BlockSpec((D, H1), lambda i: (0, 0)),
            pl.BlockSpec((NW, BN), lambda i: (0, i)),
        ],
        out_specs=[
            pl.BlockSpec((BN, H1), lambda i: (i, 0)),
            pl.BlockSpec((BN, 1), lambda i: (i, 0)),
        ],
        out_shape=[
            jax.ShapeDtypeStruct((N, H1), jnp.float32),
            jax.ShapeDtypeStruct((N, 1), jnp.float32),
        ],
    )(x, W1, degp)


def _dense2_body(p_ref, hs_ref, dinv_ref, b_ref, w_ref, out_ref):
    a = p_ref[...]
    dinv = dinv_ref[...]
    t = (a[0] + a[1] + hs_ref[...]) * dinv + b_ref[...]
    t = jnp.maximum(t, 0.0)
    hh = jnp.dot(t, w_ref[...], preferred_element_type=jnp.float32)
    out_ref[...] = hh * dinv


def _dense2(P, hs1, dinv, b1r, W2):
    return pl.pallas_call(
        _dense2_body,
        grid=(GRID,),
        in_specs=[
            pl.BlockSpec((NC, BN, H1), lambda i: (0, i, 0)),
            pl.BlockSpec((BN, H1), lambda i: (i, 0)),
            pl.BlockSpec((BN, 1), lambda i: (i, 0)),
            pl.BlockSpec((1, H1), lambda i: (0, 0)),
            pl.BlockSpec((H1, H2), lambda i: (0, 0)),
        ],
        out_specs=pl.BlockSpec((BN, H2), lambda i: (i, 0)),
        out_shape=jax.ShapeDtypeStruct((N, H2), jnp.float32),
    )(P, hs1, dinv, b1r, W2)


def _dense3_body(q_ref, hs_ref, dinv_ref, b_ref, w_ref, bfc_ref, out_ref):
    a = q_ref[...]
    dinv = dinv_ref[...]
    t = (a[0] + a[1] + hs_ref[...]) * dinv + b_ref[...]
    t = jnp.maximum(t, 0.0)
    logits = jnp.dot(t, w_ref[...], preferred_element_type=jnp.float32)
    logits = logits + bfc_ref[...]
    m = jnp.max(logits, axis=1, keepdims=True)
    e = jnp.exp(logits - m)
    out_ref[...] = e / jnp.sum(e, axis=1, keepdims=True)


def _dense3(Q, hs2, dinv, b2r, Wfcp, bfcp):
    return pl.pallas_call(
        _dense3_body,
        grid=(GRID,),
        in_specs=[
            pl.BlockSpec((NC, BN, H2), lambda i: (0, i, 0)),
            pl.BlockSpec((BN, H2), lambda i: (i, 0)),
            pl.BlockSpec((BN, 1), lambda i: (i, 0)),
            pl.BlockSpec((1, H2), lambda i: (0, 0)),
            pl.BlockSpec((H2, CP), lambda i: (0, 0)),
            pl.BlockSpec((1, CP), lambda i: (0, 0)),
        ],
        out_specs=pl.BlockSpec((BN, CP), lambda i: (i, 0)),
        out_shape=jax.ShapeDtypeStruct((N, CP), jnp.float32),
    )(Q, hs2, dinv, b2r, Wfcp, bfcp)


def kernel(x, edge_index, W1, b1, W2, b2, Wfc, bfc):
    src = edge_index[0]
    dst = edge_index[1]
    pad = jnp.full((EPAD - E,), N, dtype=jnp.int32)
    srcs = jnp.concatenate([src, pad]).reshape(NCHUNKS, K)
    dsts = jnp.concatenate([dst, pad]).reshape(NCHUNKS, K)

    zeros_deg = jnp.zeros((NP1,), jnp.float32)
    degp = _deg_kernel(dsts, zeros_deg)

    hs1, dinv = _dense1(x, W1, degp[:, :N])

    zeros1 = jnp.zeros((NPAD, H1), jnp.float32)
    hs1p = jnp.concatenate([hs1, jnp.zeros((NPAD - N, H1), jnp.float32)])
    P = _agg_h1(hs1p, srcs, dsts, zeros1)

    hs2 = _dense2(P, hs1, dinv, b1.reshape(1, H1), W2)

    zeros2 = jnp.zeros((NPAD, H2), jnp.float32)
    hs2p = jnp.concatenate([hs2, jnp.zeros((NPAD - N, H2), jnp.float32)])
    Q = _agg_h2(hs2p, srcs, dsts, zeros2)

    Wfcp = jnp.concatenate([Wfc, jnp.zeros((H2, CP - C), jnp.float32)], axis=1)
    bfcp = jnp.concatenate([bfc, jnp.full((CP - C,), -1e30, jnp.float32)])
    out = _dense3(Q, hs2, dinv, b2.reshape(1, H2), Wfcp, bfcp.reshape(1, CP))
    return out[:, :C]


# speedup vs baseline: 23.0456x; 23.0456x over previous
import functools

import jax
import jax.numpy as jnp
from jax import lax
from jax.experimental import pallas as pl
from jax.experimental.pallas import tpu as pltpu
from jax.experimental.pallas import tpu_sc as plsc

N = 10000
E = 320000
D = 128
H1 = 32
H2 = 16
C = 7

NC = 2
NS = 16
NW = NC * NS
K = 128
WCHUNKS = 80
NCHUNKS = NW * WCHUNKS
EPAD = NCHUNKS * K
N2 = 10240

BN = 1024
GRID = N2 // BN
CP = 8


def _sc_mesh():
    return plsc.VectorSubcoreMesh(core_axis_name="c", subcore_axis_name="s")


_SC_PARAMS = pltpu.CompilerParams(
    needs_layout_passes=False,
    use_tc_tiling_on_sc=False,
)


@functools.partial(
    pl.kernel,
    out_type=jax.ShapeDtypeStruct((NW, N2), jnp.float32),
    mesh=_sc_mesh(),
    compiler_params=_SC_PARAMS,
    scratch_types=[
        pltpu.VMEM((WCHUNKS, K), jnp.int32),
        pltpu.VMEM((N2,), jnp.float32),
    ],
)
def _deg_kernel(dsts_hbm, zeros_hbm, out_hbm, dst_v, deg_v):
    c = lax.axis_index("c")
    s = lax.axis_index("s")
    wid = s * NC + c
    pltpu.sync_copy(zeros_hbm, deg_v)
    pltpu.sync_copy(dsts_hbm.at[pl.ds(wid * WCHUNKS, WCHUNKS)], dst_v)
    ones = jnp.full((16,), 1.0, dtype=jnp.float32)

    def body(j, carry):
        row = dst_v.at[j]
        for g in range(K // 16):
            idx = row[pl.ds(g * 16, 16)]
            plsc.addupdate_scatter(deg_v, [idx], ones)
        return carry

    lax.fori_loop(0, WCHUNKS, body, 0)
    pltpu.sync_copy(deg_v, out_hbm.at[wid])


def _make_agg(H):
    @functools.partial(
        pl.kernel,
        out_type=jax.ShapeDtypeStruct((NC, N2, H), jnp.float32),
        mesh=_sc_mesh(),
        compiler_params=_SC_PARAMS,
        scratch_types=[
            pltpu.VMEM((WCHUNKS, K), jnp.int32),
            pltpu.VMEM((WCHUNKS, K), jnp.int32),
            pltpu.VMEM((K, H), jnp.float32),
            pltpu.VMEM_SHARED((N2, H), jnp.float32),
            pltpu.SemaphoreType.DMA,
        ],
    )
    def agg(hs_hbm, srcs_hbm, dsts_hbm, zeros_hbm, out_hbm,
            src_v, dst_v, rows_v, acc, sem):
        c = lax.axis_index("c")
        s = lax.axis_index("s")
        wid = s * NC + c

        @pl.when(s == 0)
        def _():
            pltpu.sync_copy(zeros_hbm, acc)

        pltpu.sync_copy(srcs_hbm.at[pl.ds(wid * WCHUNKS, WCHUNKS)], src_v)
        pltpu.sync_copy(dsts_hbm.at[pl.ds(wid * WCHUNKS, WCHUNKS)], dst_v)
        plsc.subcore_barrier()

        def body(j, carry):
            pltpu.async_copy(hs_hbm.at[src_v.at[j]], rows_v, sem).wait()
            pltpu.sync_copy(rows_v, acc.at[dst_v.at[j]], add=True)
            return carry

        lax.fori_loop(0, WCHUNKS, body, 0)
        plsc.subcore_barrier()

        @pl.when(s == 0)
        def _():
            pltpu.sync_copy(acc, out_hbm.at[c])

    return agg


_agg_h1 = _make_agg(H1)
_agg_h2 = _make_agg(H2)


def _dense1_body(x_ref, w_ref, degp_ref, hs_ref, dinv_ref):
    p = degp_ref[...]
    ones = jnp.ones((NW, 1), dtype=jnp.float32)
    deg = lax.dot_general(p, ones, (((0,), (0,)), ((), ())),
                          preferred_element_type=jnp.float32)
    dinv = lax.rsqrt(deg + 1.0)
    h = jnp.dot(x_ref[...], w_ref[...], preferred_element_type=jnp.float32)
    hs_ref[...] = h * dinv
    dinv_ref[...] = dinv


def _dense1(x, W1, degp):
    return pl.pallas_call(
        _dense1_body,
        grid=(GRID,),
        in_specs=[
            pl.BlockSpec((BN, D), lambda i: (i, 0)),
            pl.BlockSpec((D, H1), lambda i: (0, 0)),
            pl.BlockSpec((NW, BN), lambda i: (0, i)),
        ],
        out_specs=[
            pl.BlockSpec((BN, H1), lambda i: (i, 0)),
            pl.BlockSpec((BN, 1), lambda i: (i, 0)),
        ],
        out_shape=[
            jax.ShapeDtypeStruct((N2, H1), jnp.float32),
            jax.ShapeDtypeStruct((N2, 1), jnp.float32),
        ],
    )(x, W1, degp)


def _dense2_body(p_ref, hs_ref, dinv_ref, b_ref, w_ref, out_ref):
    a = p_ref[...]
    dinv = dinv_ref[...]
    t = (a[0] + a[1] + hs_ref[...]) * dinv + b_ref[...]
    t = jnp.maximum(t, 0.0)
    hh = jnp.dot(t, w_ref[...], preferred_element_type=jnp.float32)
    out_ref[...] = hh * dinv


def _dense2(P, hs1, dinv, b1r, W2):
    return pl.pallas_call(
        _dense2_body,
        grid=(GRID,),
        in_specs=[
            pl.BlockSpec((NC, BN, H1), lambda i: (0, i, 0)),
            pl.BlockSpec((BN, H1), lambda i: (i, 0)),
            pl.BlockSpec((BN, 1), lambda i: (i, 0)),
            pl.BlockSpec((1, H1), lambda i: (0, 0)),
            pl.BlockSpec((H1, H2), lambda i: (0, 0)),
        ],
        out_specs=pl.BlockSpec((BN, H2), lambda i: (i, 0)),
        out_shape=jax.ShapeDtypeStruct((N2, H2), jnp.float32),
    )(P, hs1, dinv, b1r, W2)


def _dense3_body(q_ref, hs_ref, dinv_ref, b_ref, w_ref, bfc_ref, out_ref):
    a = q_ref[...]
    dinv = dinv_ref[...]
    t = (a[0] + a[1] + hs_ref[...]) * dinv + b_ref[...]
    t = jnp.maximum(t, 0.0)
    logits = jnp.dot(t, w_ref[...], preferred_element_type=jnp.float32)
    logits = logits + bfc_ref[...]
    m = jnp.max(logits, axis=1, keepdims=True)
    e = jnp.exp(logits - m)
    out_ref[...] = e / jnp.sum(e, axis=1, keepdims=True)


def _dense3(Q, hs2, dinv, b2r, Wfcp, bfcp):
    return pl.pallas_call(
        _dense3_body,
        grid=(GRID,),
        in_specs=[
            pl.BlockSpec((NC, BN, H2), lambda i: (0, i, 0)),
            pl.BlockSpec((BN, H2), lambda i: (i, 0)),
            pl.BlockSpec((BN, 1), lambda i: (i, 0)),
            pl.BlockSpec((1, H2), lambda i: (0, 0)),
            pl.BlockSpec((H2, CP), lambda i: (0, 0)),
            pl.BlockSpec((1, CP), lambda i: (0, 0)),
        ],
        out_specs=pl.BlockSpec((BN, CP), lambda i: (i, 0)),
        out_shape=jax.ShapeDtypeStruct((N2, CP), jnp.float32),
    )(Q, hs2, dinv, b2r, Wfcp, bfcp)


def kernel(x, edge_index, W1, b1, W2, b2, Wfc, bfc):
    src = edge_index[0]
    dst = edge_index[1]
    pad = jnp.full((EPAD - E,), N, dtype=jnp.int32)
    srcs = jnp.concatenate([src, pad]).reshape(NCHUNKS, K)
    dsts = jnp.concatenate([dst, pad]).reshape(NCHUNKS, K)

    zeros_deg = jnp.zeros((N2,), jnp.float32)
    degp = _deg_kernel(dsts, zeros_deg)

    hs1, dinv = _dense1(x, W1, degp)

    zeros1 = jnp.zeros((N2, H1), jnp.float32)
    P = _agg_h1(hs1, srcs, dsts, zeros1)

    hs2 = _dense2(P, hs1, dinv, b1.reshape(1, H1), W2)

    zeros2 = jnp.zeros((N2, H2), jnp.float32)
    Q = _agg_h2(hs2, srcs, dsts, zeros2)

    Wfcp = jnp.concatenate([Wfc, jnp.zeros((H2, CP - C), jnp.float32)], axis=1)
    bfcp = jnp.concatenate([bfc, jnp.full((CP - C,), -1e30, jnp.float32)])
    out = _dense3(Q, hs2, dinv, b2.reshape(1, H2), Wfcp, bfcp.reshape(1, CP))
    return out[:N, :C]


# speedup vs baseline: 28.8031x; 1.2498x over previous
import functools

import jax
import jax.numpy as jnp
from jax import lax
from jax.experimental import pallas as pl
from jax.experimental.pallas import tpu as pltpu
from jax.experimental.pallas import tpu_sc as plsc

N = 10000
E = 320000
D = 128
H1 = 32
H2 = 16
C = 7

NC = 2
NS = 16
NW = NC * NS
K = 128
WCHUNKS = 80
NCHUNKS = NW * WCHUNKS
EPAD = NCHUNKS * K
N2 = 10240

BN = 1024
GRID = N2 // BN
CP = 8


def _sc_mesh():
    return plsc.VectorSubcoreMesh(
        core_axis_name="c", subcore_axis_name="s",
        num_cores=NC, num_subcores=NS,
    )


_SC_PARAMS = pltpu.CompilerParams(
    needs_layout_passes=False,
    use_tc_tiling_on_sc=False,
)


@functools.partial(
    pl.kernel,
    out_type=jax.ShapeDtypeStruct((NW, N2), jnp.float32),
    mesh=_sc_mesh(),
    compiler_params=_SC_PARAMS,
    scratch_types=[
        pltpu.VMEM((WCHUNKS, K), jnp.int32),
        pltpu.VMEM((N2,), jnp.float32),
    ],
)
def _deg_kernel(dsts_hbm, zeros_hbm, out_hbm, dst_v, deg_v):
    c = lax.axis_index("c")
    s = lax.axis_index("s")
    wid = s * NC + c
    pltpu.sync_copy(zeros_hbm, deg_v)
    pltpu.sync_copy(dsts_hbm.at[pl.ds(wid * WCHUNKS, WCHUNKS)], dst_v)
    ones = jnp.full((16,), 1.0, dtype=jnp.float32)

    def body(j, carry):
        row = dst_v.at[j]
        for g in range(K // 16):
            idx = row[pl.ds(g * 16, 16)]
            plsc.addupdate_scatter(deg_v, [idx], ones)
        return carry

    lax.fori_loop(0, WCHUNKS, body, 0)
    pltpu.sync_copy(deg_v, out_hbm.at[wid])


def _make_agg(H):
    @functools.partial(
        pl.kernel,
        out_type=jax.ShapeDtypeStruct((NC, N2, H), jnp.float32),
        mesh=_sc_mesh(),
        compiler_params=_SC_PARAMS,
        scratch_types=[
            pltpu.VMEM((WCHUNKS, K), jnp.int32),
            pltpu.VMEM((WCHUNKS, K), jnp.int32),
            pltpu.VMEM((K, H), jnp.float32),
            pltpu.VMEM((K, H), jnp.float32),
            pltpu.VMEM_SHARED((N2, H), jnp.float32),
            pltpu.SemaphoreType.DMA,
            pltpu.SemaphoreType.DMA,
        ],
    )
    def agg(hs_hbm, srcs_hbm, dsts_hbm, zeros_hbm, out_hbm,
            src_v, dst_v, rows0, rows1, acc, sem0, sem1):
        c = lax.axis_index("c")
        s = lax.axis_index("s")
        wid = s * NC + c

        @pl.when(s == 0)
        def _():
            pltpu.sync_copy(zeros_hbm, acc)

        pltpu.sync_copy(srcs_hbm.at[pl.ds(wid * WCHUNKS, WCHUNKS)], src_v)
        pltpu.sync_copy(dsts_hbm.at[pl.ds(wid * WCHUNKS, WCHUNKS)], dst_v)
        plsc.subcore_barrier()

        pltpu.async_copy(hs_hbm.at[src_v.at[0]], rows0, sem0)

        def body(i, carry):
            j0 = 2 * i
            j1 = 2 * i + 1
            pltpu.async_copy(hs_hbm.at[src_v.at[j1]], rows1, sem1)
            pltpu.make_async_copy(hs_hbm.at[src_v.at[j0]], rows0, sem0).wait()
            pltpu.sync_copy(rows0, acc.at[dst_v.at[j0]], add=True)
            jn = jnp.minimum(j1 + 1, WCHUNKS - 1)
            pltpu.async_copy(hs_hbm.at[src_v.at[jn]], rows0, sem0)
            pltpu.make_async_copy(hs_hbm.at[src_v.at[j1]], rows1, sem1).wait()
            pltpu.sync_copy(rows1, acc.at[dst_v.at[j1]], add=True)
            return carry

        lax.fori_loop(0, WCHUNKS // 2, body, 0)
        pltpu.make_async_copy(hs_hbm.at[src_v.at[0]], rows0, sem0).wait()
        plsc.subcore_barrier()

        @pl.when(s == 0)
        def _():
            pltpu.sync_copy(acc, out_hbm.at[c])

    return agg


_agg_h1 = _make_agg(H1)
_agg_h2 = _make_agg(H2)


def _dense1_body(x_ref, w_ref, degp_ref, hs_ref, dinv_ref):
    p = degp_ref[...]
    ones = jnp.ones((NW, 1), dtype=jnp.float32)
    deg = lax.dot_general(p, ones, (((0,), (0,)), ((), ())),
                          preferred_element_type=jnp.float32)
    dinv = lax.rsqrt(deg + 1.0)
    h = jnp.dot(x_ref[...], w_ref[...], preferred_element_type=jnp.float32)
    hs_ref[...] = h * dinv
    dinv_ref[...] = dinv


def _dense1(x, W1, degp):
    return pl.pallas_call(
        _dense1_body,
        grid=(GRID,),
        in_specs=[
            pl.BlockSpec((BN, D), lambda i: (i, 0)),
            pl.BlockSpec((D, H1), lambda i: (0, 0)),
            pl.BlockSpec((NW, BN), lambda i: (0, i)),
        ],
        out_specs=[
            pl.BlockSpec((BN, H1), lambda i: (i, 0)),
            pl.BlockSpec((BN, 1), lambda i: (i, 0)),
        ],
        out_shape=[
            jax.ShapeDtypeStruct((N2, H1), jnp.float32),
            jax.ShapeDtypeStruct((N2, 1), jnp.float32),
        ],
    )(x, W1, degp)


def _dense2_body(p_ref, hs_ref, dinv_ref, b_ref, w_ref, out_ref):
    a = p_ref[...]
    dinv = dinv_ref[...]
    t = (a[0] + a[1] + hs_ref[...]) * dinv + b_ref[...]
    t = jnp.maximum(t, 0.0)
    hh = jnp.dot(t, w_ref[...], preferred_element_type=jnp.float32)
    out_ref[...] = hh * dinv


def _dense2(P, hs1, dinv, b1r, W2):
    return pl.pallas_call(
        _dense2_body,
        grid=(GRID,),
        in_specs=[
            pl.BlockSpec((NC, BN, H1), lambda i: (0, i, 0)),
            pl.BlockSpec((BN, H1), lambda i: (i, 0)),
            pl.BlockSpec((BN, 1), lambda i: (i, 0)),
            pl.BlockSpec((1, H1), lambda i: (0, 0)),
            pl.BlockSpec((H1, H2), lambda i: (0, 0)),
        ],
        out_specs=pl.BlockSpec((BN, H2), lambda i: (i, 0)),
        out_shape=jax.ShapeDtypeStruct((N2, H2), jnp.float32),
    )(P, hs1, dinv, b1r, W2)


def _dense3_body(q_ref, hs_ref, dinv_ref, b_ref, w_ref, bfc_ref, out_ref):
    a = q_ref[...]
    dinv = dinv_ref[...]
    t = (a[0] + a[1] + hs_ref[...]) * dinv + b_ref[...]
    t = jnp.maximum(t, 0.0)
    logits = jnp.dot(t, w_ref[...], preferred_element_type=jnp.float32)
    logits = logits + bfc_ref[...]
    m = jnp.max(logits, axis=1, keepdims=True)
    e = jnp.exp(logits - m)
    out_ref[...] = e / jnp.sum(e, axis=1, keepdims=True)


def _dense3(Q, hs2, dinv, b2r, Wfcp, bfcp):
    return pl.pallas_call(
        _dense3_body,
        grid=(GRID,),
        in_specs=[
            pl.BlockSpec((NC, BN, H2), lambda i: (0, i, 0)),
            pl.BlockSpec((BN, H2), lambda i: (i, 0)),
            pl.BlockSpec((BN, 1), lambda i: (i, 0)),
            pl.BlockSpec((1, H2), lambda i: (0, 0)),
            pl.BlockSpec((H2, CP), lambda i: (0, 0)),
            pl.BlockSpec((1, CP), lambda i: (0, 0)),
        ],
        out_specs=pl.BlockSpec((BN, CP), lambda i: (i, 0)),
        out_shape=jax.ShapeDtypeStruct((N2, CP), jnp.float32),
    )(Q, hs2, dinv, b2r, Wfcp, bfcp)


def kernel(x, edge_index, W1, b1, W2, b2, Wfc, bfc):
    src = edge_index[0]
    dst = edge_index[1]
    pad = jnp.full((EPAD - E,), N, dtype=jnp.int32)
    srcs = jnp.concatenate([src, pad]).reshape(NCHUNKS, K)
    dsts = jnp.concatenate([dst, pad]).reshape(NCHUNKS, K)

    zeros_deg = jnp.zeros((N2,), jnp.float32)
    degp = _deg_kernel(dsts, zeros_deg)

    hs1, dinv = _dense1(x, W1, degp)

    zeros1 = jnp.zeros((N2, H1), jnp.float32)
    P = _agg_h1(hs1, srcs, dsts, zeros1)

    hs2 = _dense2(P, hs1, dinv, b1.reshape(1, H1), W2)

    zeros2 = jnp.zeros((N2, H2), jnp.float32)
    Q = _agg_h2(hs2, srcs, dsts, zeros2)

    Wfcp = jnp.concatenate([Wfc, jnp.zeros((H2, CP - C), jnp.float32)], axis=1)
    bfcp = jnp.concatenate([bfc, jnp.full((CP - C,), -1e30, jnp.float32)])
    out = _dense3(Q, hs2, dinv, b2.reshape(1, H2), Wfcp, bfcp.reshape(1, CP))
    return out[:N, :C]


# speedup vs baseline: 40.2738x; 1.3982x over previous
import functools

import jax
import jax.numpy as jnp
from jax import lax
from jax.experimental import pallas as pl
from jax.experimental.pallas import tpu as pltpu
from jax.experimental.pallas import tpu_sc as plsc

N = 10000
E = 320000
D = 128
H1 = 32
H2 = 16
C = 7

NC = 2
NS = 16
NW = NC * NS
K = 128
NCHUNKS = E // K
CA = 110
CB = 46
SLAB = CA + 1
N2 = 10240

BN = 1024
GRID = N2 // BN
CP = 8


def _sc_mesh():
    return plsc.VectorSubcoreMesh(
        core_axis_name="c", subcore_axis_name="s",
        num_cores=NC, num_subcores=NS,
    )


_SC_PARAMS = pltpu.CompilerParams(
    needs_layout_passes=False,
    use_tc_tiling_on_sc=False,
)


def _agg_split(c, s):
    is0 = c == 0
    count = jnp.where(is0, CA + (s < 4).astype(jnp.int32), CB)
    start = jnp.where(
        is0,
        s * CA + jnp.minimum(s, 4),
        16 * CA + 4 + s * CB,
    )
    dma_start = jnp.minimum(start, NCHUNKS - SLAB)
    return count, dma_start, start - dma_start


def _deg_split(wid):
    count = 78 + (wid < 4).astype(jnp.int32)
    start = wid * 78 + jnp.minimum(wid, 4)
    dma_start = jnp.minimum(start, NCHUNKS - 79)
    return count, dma_start, start - dma_start


@functools.partial(
    pl.kernel,
    out_type=jax.ShapeDtypeStruct((NW, N2), jnp.float32),
    mesh=_sc_mesh(),
    compiler_params=_SC_PARAMS,
    scratch_types=[
        pltpu.VMEM((79, K), jnp.int32),
        pltpu.VMEM((N2,), jnp.float32),
    ],
)
def _deg_kernel(dsts_hbm, zeros_hbm, out_hbm, dst_v, deg_v):
    c = lax.axis_index("c")
    s = lax.axis_index("s")
    wid = s * NC + c
    count, dma_start, off = _deg_split(wid)
    pltpu.sync_copy(zeros_hbm, deg_v)
    pltpu.sync_copy(dsts_hbm.at[pl.ds(dma_start, 79)], dst_v)
    ones = jnp.full((16,), 1.0, dtype=jnp.float32)

    def body(j, carry):
        row = dst_v.at[off + j]
        for g in range(K // 16):
            idx = row[pl.ds(g * 16, 16)]
            plsc.addupdate_scatter(deg_v, [idx], ones)
        return carry

    lax.fori_loop(0, count, body, 0)
    pltpu.sync_copy(deg_v, out_hbm.at[wid])


def _make_agg(H):
    @functools.partial(
        pl.kernel,
        out_type=jax.ShapeDtypeStruct((NC, N2, H), jnp.float32),
        mesh=_sc_mesh(),
        compiler_params=_SC_PARAMS,
        scratch_types=[
            pltpu.VMEM((SLAB, K), jnp.int32),
            pltpu.VMEM((SLAB, K), jnp.int32),
            pltpu.VMEM((K, H), jnp.float32),
            pltpu.VMEM((K, H), jnp.float32),
            pltpu.VMEM_SHARED((N2, H), jnp.float32),
            pltpu.SemaphoreType.DMA,
            pltpu.SemaphoreType.DMA,
        ],
    )
    def agg(hs_hbm, srcs_hbm, dsts_hbm, zeros_hbm, out_hbm,
            src_v, dst_v, rows0, rows1, acc, sem0, sem1):
        c = lax.axis_index("c")
        s = lax.axis_index("s")
        count, dma_start, off = _agg_split(c, s)

        @pl.when(s == 0)
        def _():
            pltpu.sync_copy(zeros_hbm, acc)

        pltpu.sync_copy(srcs_hbm.at[pl.ds(dma_start, SLAB)], src_v)
        pltpu.sync_copy(dsts_hbm.at[pl.ds(dma_start, SLAB)], dst_v)
        plsc.subcore_barrier()

        pltpu.async_copy(hs_hbm.at[src_v.at[off]], rows0, sem0)

        def body(i, carry):
            j0 = off + 2 * i
            j1 = j0 + 1
            pltpu.async_copy(hs_hbm.at[src_v.at[j1]], rows1, sem1)
            pltpu.make_async_copy(hs_hbm.at[src_v.at[j0]], rows0, sem0).wait()
            pltpu.sync_copy(rows0, acc.at[dst_v.at[j0]], add=True)
            jn = jnp.minimum(j0 + 2, off + count - 1)
            pltpu.async_copy(hs_hbm.at[src_v.at[jn]], rows0, sem0)
            pltpu.make_async_copy(hs_hbm.at[src_v.at[j1]], rows1, sem1).wait()
            pltpu.sync_copy(rows1, acc.at[dst_v.at[j1]], add=True)
            return carry

        lax.fori_loop(0, count // 2, body, 0)
        pltpu.make_async_copy(hs_hbm.at[src_v.at[0]], rows0, sem0).wait()

        @pl.when(count % 2 == 1)
        def _():
            pltpu.sync_copy(rows0, acc.at[dst_v.at[off + count - 1]], add=True)

        plsc.subcore_barrier()

        @pl.when(s == 0)
        def _():
            pltpu.sync_copy(acc, out_hbm.at[c])

    return agg


_agg_h1 = _make_agg(H1)
_agg_h2 = _make_agg(H2)


def _dense1_body(x_ref, w_ref, degp_ref, hs_ref, dinv_ref):
    p = degp_ref[...]
    ones = jnp.ones((NW, 1), dtype=jnp.float32)
    deg = lax.dot_general(p, ones, (((0,), (0,)), ((), ())),
                          preferred_element_type=jnp.float32)
    dinv = lax.rsqrt(deg + 1.0)
    h = jnp.dot(x_ref[...], w_ref[...], preferred_element_type=jnp.float32)
    hs_ref[...] = h * dinv
    dinv_ref[...] = dinv


def _dense1(x, W1, degp):
    return pl.pallas_call(
        _dense1_body,
        grid=(GRID,),
        in_specs=[
            pl.BlockSpec((BN, D), lambda i: (i, 0)),
            pl.BlockSpec((D, H1), lambda i: (0, 0)),
            pl.BlockSpec((NW, BN), lambda i: (0, i)),
        ],
        out_specs=[
            pl.BlockSpec((BN, H1), lambda i: (i, 0)),
            pl.BlockSpec((BN, 1), lambda i: (i, 0)),
        ],
        out_shape=[
            jax.ShapeDtypeStruct((N2, H1), jnp.float32),
            jax.ShapeDtypeStruct((N2, 1), jnp.float32),
        ],
    )(x, W1, degp)


def _dense2_body(p_ref, hs_ref, dinv_ref, b_ref, w_ref, out_ref):
    a = p_ref[...]
    dinv = dinv_ref[...]
    t = (a[0] + a[1] + hs_ref[...]) * dinv + b_ref[...]
    t = jnp.maximum(t, 0.0)
    hh = jnp.dot(t, w_ref[...], preferred_element_type=jnp.float32)
    out_ref[...] = hh * dinv


def _dense2(P, hs1, dinv, b1r, W2):
    return pl.pallas_call(
        _dense2_body,
        grid=(GRID,),
        in_specs=[
            pl.BlockSpec((NC, BN, H1), lambda i: (0, i, 0)),
            pl.BlockSpec((BN, H1), lambda i: (i, 0)),
            pl.BlockSpec((BN, 1), lambda i: (i, 0)),
            pl.BlockSpec((1, H1), lambda i: (0, 0)),
            pl.BlockSpec((H1, H2), lambda i: (0, 0)),
        ],
        out_specs=pl.BlockSpec((BN, H2), lambda i: (i, 0)),
        out_shape=jax.ShapeDtypeStruct((N2, H2), jnp.float32),
    )(P, hs1, dinv, b1r, W2)


def _dense3_body(q_ref, hs_ref, dinv_ref, b_ref, w_ref, bfc_ref, out_ref):
    a = q_ref[...]
    dinv = dinv_ref[...]
    t = (a[0] + a[1] + hs_ref[...]) * dinv + b_ref[...]
    t = jnp.maximum(t, 0.0)
    logits = jnp.dot(t, w_ref[...], preferred_element_type=jnp.float32)
    logits = logits + bfc_ref[...]
    m = jnp.max(logits, axis=1, keepdims=True)
    e = jnp.exp(logits - m)
    out_ref[...] = e / jnp.sum(e, axis=1, keepdims=True)


def _dense3(Q, hs2, dinv, b2r, Wfcp, bfcp):
    return pl.pallas_call(
        _dense3_body,
        grid=(GRID,),
        in_specs=[
            pl.BlockSpec((NC, BN, H2), lambda i: (0, i, 0)),
            pl.BlockSpec((BN, H2), lambda i: (i, 0)),
            pl.BlockSpec((BN, 1), lambda i: (i, 0)),
            pl.BlockSpec((1, H2), lambda i: (0, 0)),
            pl.BlockSpec((H2, CP), lambda i: (0, 0)),
            pl.BlockSpec((1, CP), lambda i: (0, 0)),
        ],
        out_specs=pl.BlockSpec((BN, CP), lambda i: (i, 0)),
        out_shape=jax.ShapeDtypeStruct((N2, CP), jnp.float32),
    )(Q, hs2, dinv, b2r, Wfcp, bfcp)


def kernel(x, edge_index, W1, b1, W2, b2, Wfc, bfc):
    srcs = edge_index[0].reshape(NCHUNKS, K)
    dsts = edge_index[1].reshape(NCHUNKS, K)

    zeros_deg = jnp.zeros((N2,), jnp.float32)
    degp = _deg_kernel(dsts, zeros_deg)

    hs1, dinv = _dense1(x, W1, degp)

    zeros1 = jnp.zeros((N2, H1), jnp.float32)
    P = _agg_h1(hs1, srcs, dsts, zeros1)

    hs2 = _dense2(P, hs1, dinv, b1.reshape(1, H1), W2)

    zeros2 = jnp.zeros((N2, H2), jnp.float32)
    Q = _agg_h2(hs2, srcs, dsts, zeros2)

    Wfcp = jnp.concatenate([Wfc, jnp.zeros((H2, CP - C), jnp.float32)], axis=1)
    bfcp = jnp.concatenate([bfc, jnp.full((CP - C,), -1e30, jnp.float32)])
    out = _dense3(Q, hs2, dinv, b2.reshape(1, H2), Wfcp, bfcp.reshape(1, CP))
    return out[:N, :C]


# speedup vs baseline: 44.2328x; 1.0983x over previous
import functools

import jax
import jax.numpy as jnp
from jax import lax
from jax.experimental import pallas as pl
from jax.experimental.pallas import tpu as pltpu
from jax.experimental.pallas import tpu_sc as plsc

N = 10000
E = 320000
D = 128
H1 = 32
H2 = 16
C = 7

NC = 2
NS = 16
NW = NC * NS
K = 128
NCHUNKS = E // K
CA = 100
CB = 56
SLAB = CA + 1
N2 = 10240

BN = 1024
GRID = N2 // BN
CP = 8


def _sc_mesh():
    return plsc.VectorSubcoreMesh(
        core_axis_name="c", subcore_axis_name="s",
        num_cores=NC, num_subcores=NS,
    )


_SC_PARAMS = pltpu.CompilerParams(
    needs_layout_passes=False,
    use_tc_tiling_on_sc=False,
)


def _agg_split(c, s):
    is0 = c == 0
    count = jnp.where(is0, CA + (s < 4).astype(jnp.int32), CB)
    start = jnp.where(
        is0,
        s * CA + jnp.minimum(s, 4),
        16 * CA + 4 + s * CB,
    )
    dma_start = jnp.minimum(start, NCHUNKS - SLAB)
    return count, dma_start, start - dma_start


def _deg_split(wid):
    count = 78 + (wid < 4).astype(jnp.int32)
    start = wid * 78 + jnp.minimum(wid, 4)
    dma_start = jnp.minimum(start, NCHUNKS - 79)
    return count, dma_start, start - dma_start


@functools.partial(
    pl.kernel,
    out_type=jax.ShapeDtypeStruct((NW, N2), jnp.float32),
    mesh=_sc_mesh(),
    compiler_params=_SC_PARAMS,
    scratch_types=[
        pltpu.VMEM((79, K), jnp.int32),
        pltpu.VMEM((N2,), jnp.float32),
    ],
)
def _deg_kernel(ei_hbm, zeros_hbm, out_hbm, dst_v, deg_v):
    c = lax.axis_index("c")
    s = lax.axis_index("s")
    wid = s * NC + c
    count, dma_start, off = _deg_split(wid)
    pltpu.sync_copy(zeros_hbm, deg_v)
    pltpu.sync_copy(ei_hbm.at[1].at[pl.ds(dma_start, 79)], dst_v)
    ones = jnp.full((16,), 1.0, dtype=jnp.float32)

    def body(j, carry):
        row = dst_v.at[off + j]
        for g in range(K // 16):
            idx = row[pl.ds(g * 16, 16)]
            plsc.addupdate_scatter(deg_v, [idx], ones)
        return carry

    lax.fori_loop(0, count, body, 0)
    pltpu.sync_copy(deg_v, out_hbm.at[wid])


def _make_agg(H):
    @functools.partial(
        pl.kernel,
        out_type=jax.ShapeDtypeStruct((NC, N2, H), jnp.float32),
        mesh=_sc_mesh(),
        compiler_params=_SC_PARAMS,
        scratch_types=[
            pltpu.VMEM((SLAB, K), jnp.int32),
            pltpu.VMEM((SLAB, K), jnp.int32),
            pltpu.VMEM((K, H), jnp.float32),
            pltpu.VMEM((K, H), jnp.float32),
            pltpu.VMEM_SHARED((N2, H), jnp.float32),
            pltpu.SemaphoreType.DMA,
            pltpu.SemaphoreType.DMA,
        ],
    )
    def agg(hs_hbm, ei_hbm, zeros_hbm, out_hbm,
            src_v, dst_v, rows0, rows1, acc, sem0, sem1):
        c = lax.axis_index("c")
        s = lax.axis_index("s")
        count, dma_start, off = _agg_split(c, s)

        @pl.when(s == 0)
        def _():
            pltpu.sync_copy(zeros_hbm, acc)

        pltpu.sync_copy(ei_hbm.at[0].at[pl.ds(dma_start, SLAB)], src_v)
        pltpu.sync_copy(ei_hbm.at[1].at[pl.ds(dma_start, SLAB)], dst_v)
        plsc.subcore_barrier()

        pltpu.async_copy(hs_hbm.at[src_v.at[off]], rows0, sem0)

        def body(i, carry):
            j0 = off + 2 * i
            j1 = j0 + 1
            pltpu.async_copy(hs_hbm.at[src_v.at[j1]], rows1, sem1)
            pltpu.make_async_copy(hs_hbm.at[src_v.at[j0]], rows0, sem0).wait()
            pltpu.sync_copy(rows0, acc.at[dst_v.at[j0]], add=True)
            jn = jnp.minimum(j0 + 2, off + count - 1)
            pltpu.async_copy(hs_hbm.at[src_v.at[jn]], rows0, sem0)
            pltpu.make_async_copy(hs_hbm.at[src_v.at[j1]], rows1, sem1).wait()
            pltpu.sync_copy(rows1, acc.at[dst_v.at[j1]], add=True)
            return carry

        lax.fori_loop(0, count // 2, body, 0)
        pltpu.make_async_copy(hs_hbm.at[src_v.at[0]], rows0, sem0).wait()

        @pl.when(count % 2 == 1)
        def _():
            pltpu.sync_copy(rows0, acc.at[dst_v.at[off + count - 1]], add=True)

        plsc.subcore_barrier()

        @pl.when(s == 0)
        def _():
            pltpu.sync_copy(acc, out_hbm.at[c])

    return agg


_agg_h1 = _make_agg(H1)
_agg_h2 = _make_agg(H2)


_DEG_DOT = (((0,), (0,)), ((), ()))


def _dinv_col(degp_block):
    ones = jnp.ones((NW, 1), dtype=jnp.float32)
    deg = lax.dot_general(degp_block, ones, _DEG_DOT,
                          preferred_element_type=jnp.float32)
    return lax.rsqrt(deg + 1.0)


def _dense1_body(x_ref, w_ref, degp_ref, hs_ref):
    dinv = _dinv_col(degp_ref[...])
    h = jnp.dot(x_ref[...], w_ref[...], preferred_element_type=jnp.float32)
    hs_ref[...] = h * dinv


def _dense1(x, W1, degp):
    return pl.pallas_call(
        _dense1_body,
        grid=(GRID,),
        in_specs=[
            pl.BlockSpec((BN, D), lambda i: (i, 0)),
            pl.BlockSpec((D, H1), lambda i: (0, 0)),
            pl.BlockSpec((NW, BN), lambda i: (0, i)),
        ],
        out_specs=pl.BlockSpec((BN, H1), lambda i: (i, 0)),
        out_shape=jax.ShapeDtypeStruct((N2, H1), jnp.float32),
    )(x, W1, degp)


def _dense2_body(p_ref, hs_ref, degp_ref, b_ref, w_ref, out_ref):
    a = p_ref[...]
    dinv = _dinv_col(degp_ref[...])
    t = (a[0] + a[1] + hs_ref[...]) * dinv + b_ref[...]
    t = jnp.maximum(t, 0.0)
    hh = jnp.dot(t, w_ref[...], preferred_element_type=jnp.float32)
    out_ref[...] = hh * dinv


def _dense2(P, hs1, degp, b1r, W2):
    return pl.pallas_call(
        _dense2_body,
        grid=(GRID,),
        in_specs=[
            pl.BlockSpec((NC, BN, H1), lambda i: (0, i, 0)),
            pl.BlockSpec((BN, H1), lambda i: (i, 0)),
            pl.BlockSpec((NW, BN), lambda i: (0, i)),
            pl.BlockSpec((1, H1), lambda i: (0, 0)),
            pl.BlockSpec((H1, H2), lambda i: (0, 0)),
        ],
        out_specs=pl.BlockSpec((BN, H2), lambda i: (i, 0)),
        out_shape=jax.ShapeDtypeStruct((N2, H2), jnp.float32),
    )(P, hs1, degp, b1r, W2)


def _dense3_body(q_ref, hs_ref, degp_ref, b_ref, w_ref, bfc_ref, out_ref):
    a = q_ref[...]
    dinv = _dinv_col(degp_ref[...])
    t = (a[0] + a[1] + hs_ref[...]) * dinv + b_ref[...]
    t = jnp.maximum(t, 0.0)
    logits = jnp.dot(t, w_ref[...], preferred_element_type=jnp.float32)
    logits = logits + bfc_ref[...]
    m = jnp.max(logits, axis=1, keepdims=True)
    e = jnp.exp(logits - m)
    out_ref[...] = e / jnp.sum(e, axis=1, keepdims=True)


def _dense3(Q, hs2, degp, b2r, Wfcp, bfcp):
    return pl.pallas_call(
        _dense3_body,
        grid=(GRID,),
        in_specs=[
            pl.BlockSpec((NC, BN, H2), lambda i: (0, i, 0)),
            pl.BlockSpec((BN, H2), lambda i: (i, 0)),
            pl.BlockSpec((NW, BN), lambda i: (0, i)),
            pl.BlockSpec((1, H2), lambda i: (0, 0)),
            pl.BlockSpec((H2, CP), lambda i: (0, 0)),
            pl.BlockSpec((1, CP), lambda i: (0, 0)),
        ],
        out_specs=pl.BlockSpec((BN, CP), lambda i: (i, 0)),
        out_shape=jax.ShapeDtypeStruct((N2, CP), jnp.float32),
    )(Q, hs2, degp, b2r, Wfcp, bfcp)


def kernel(x, edge_index, W1, b1, W2, b2, Wfc, bfc):
    ei3 = edge_index.reshape(2, NCHUNKS, K)

    zeros_deg = jnp.zeros((N2,), jnp.float32)
    degp = _deg_kernel(ei3, zeros_deg)

    hs1 = _dense1(x, W1, degp)

    zeros1 = jnp.zeros((N2, H1), jnp.float32)
    P = _agg_h1(hs1, ei3, zeros1)

    hs2 = _dense2(P, hs1, degp, b1.reshape(1, H1), W2)

    zeros2 = jnp.zeros((N2, H2), jnp.float32)
    Q = _agg_h2(hs2, ei3, zeros2)

    Wfcp = jnp.concatenate([Wfc, jnp.zeros((H2, CP - C), jnp.float32)], axis=1)
    bfcp = jnp.concatenate([bfc, jnp.full((CP - C,), -1e30, jnp.float32)])
    out = _dense3(Q, hs2, degp, b2.reshape(1, H2), Wfcp, bfcp.reshape(1, CP))
    return out[:N, :C]


# speedup vs baseline: 48.4354x; 1.0950x over previous
import functools

import jax
import jax.numpy as jnp
from jax import lax
from jax.experimental import pallas as pl
from jax.experimental.pallas import tpu as pltpu
from jax.experimental.pallas import tpu_sc as plsc

N = 10000
E = 320000
D = 128
H1 = 32
H2 = 16
C = 7

NC = 2
NS = 16
NW = NC * NS
K = 128
NCHUNKS = E // K
CA = 88
CB = 68
SLAB = CA + 1
CP = 8


def _sc_mesh():
    return plsc.VectorSubcoreMesh(
        core_axis_name="c", subcore_axis_name="s",
        num_cores=NC, num_subcores=NS,
    )


_SC_PARAMS = pltpu.CompilerParams(
    needs_layout_passes=False,
    use_tc_tiling_on_sc=False,
)


def _agg_split(c, s):
    is0 = c == 0
    count = jnp.where(is0, CA + (s < 4).astype(jnp.int32), CB)
    start = jnp.where(
        is0,
        s * CA + jnp.minimum(s, 4),
        16 * CA + 4 + s * CB,
    )
    dma_start = jnp.minimum(start, NCHUNKS - SLAB)
    return count, dma_start, start - dma_start


def _deg_split(wid):
    count = 78 + (wid < 4).astype(jnp.int32)
    start = wid * 78 + jnp.minimum(wid, 4)
    dma_start = jnp.minimum(start, NCHUNKS - 79)
    return count, dma_start, start - dma_start


@functools.partial(
    pl.kernel,
    out_type=jax.ShapeDtypeStruct((NW, N), jnp.float32),
    mesh=_sc_mesh(),
    compiler_params=_SC_PARAMS,
    scratch_types=[
        pltpu.VMEM((79, K), jnp.int32),
        pltpu.VMEM((N,), jnp.float32),
    ],
)
def _deg_kernel(ei_hbm, zeros_hbm, out_hbm, dst_v, deg_v):
    c = lax.axis_index("c")
    s = lax.axis_index("s")
    wid = s * NC + c
    count, dma_start, off = _deg_split(wid)
    pltpu.sync_copy(zeros_hbm, deg_v)
    pltpu.sync_copy(ei_hbm.at[1].at[pl.ds(dma_start, 79)], dst_v)
    ones = jnp.full((16,), 1.0, dtype=jnp.float32)

    def body(j, carry):
        row = dst_v.at[off + j]
        for g in range(K // 16):
            idx = row[pl.ds(g * 16, 16)]
            plsc.addupdate_scatter(deg_v, [idx], ones)
        return carry

    lax.fori_loop(0, count, body, 0)
    pltpu.sync_copy(deg_v, out_hbm.at[wid])


def _make_agg(H):
    @functools.partial(
        pl.kernel,
        out_type=jax.ShapeDtypeStruct((NC, N, H), jnp.float32),
        mesh=_sc_mesh(),
        compiler_params=_SC_PARAMS,
        scratch_types=[
            pltpu.VMEM((SLAB, K), jnp.int32),
            pltpu.VMEM((SLAB, K), jnp.int32),
            pltpu.VMEM((K, H), jnp.float32),
            pltpu.VMEM((K, H), jnp.float32),
            pltpu.VMEM_SHARED((N, H), jnp.float32),
            pltpu.SemaphoreType.DMA,
            pltpu.SemaphoreType.DMA,
        ],
    )
    def agg(hs_hbm, ei_hbm, zeros_hbm, out_hbm,
            src_v, dst_v, rows0, rows1, acc, sem0, sem1):
        c = lax.axis_index("c")
        s = lax.axis_index("s")
        count, dma_start, off = _agg_split(c, s)

        @pl.when(s == 0)
        def _():
            pltpu.sync_copy(zeros_hbm, acc)

        pltpu.sync_copy(ei_hbm.at[0].at[pl.ds(dma_start, SLAB)], src_v)
        pltpu.sync_copy(ei_hbm.at[1].at[pl.ds(dma_start, SLAB)], dst_v)
        plsc.subcore_barrier()

        pltpu.async_copy(hs_hbm.at[src_v.at[off]], rows0, sem0)

        def body(i, carry):
            j0 = off + 2 * i
            j1 = j0 + 1
            pltpu.async_copy(hs_hbm.at[src_v.at[j1]], rows1, sem1)
            pltpu.make_async_copy(hs_hbm.at[src_v.at[j0]], rows0, sem0).wait()
            pltpu.sync_copy(rows0, acc.at[dst_v.at[j0]], add=True)
            jn = jnp.minimum(j0 + 2, off + count - 1)
            pltpu.async_copy(hs_hbm.at[src_v.at[jn]], rows0, sem0)
            pltpu.make_async_copy(hs_hbm.at[src_v.at[j1]], rows1, sem1).wait()
            pltpu.sync_copy(rows1, acc.at[dst_v.at[j1]], add=True)
            return carry

        lax.fori_loop(0, count // 2, body, 0)
        pltpu.make_async_copy(hs_hbm.at[src_v.at[0]], rows0, sem0).wait()

        @pl.when(count % 2 == 1)
        def _():
            pltpu.sync_copy(rows0, acc.at[dst_v.at[off + count - 1]], add=True)

        plsc.subcore_barrier()

        @pl.when(s == 0)
        def _():
            pltpu.sync_copy(acc, out_hbm.at[c])

    return agg


_agg_h1 = _make_agg(H1)
_agg_h2 = _make_agg(H2)


_DEG_DOT = (((0,), (0,)), ((), ()))


def _dinv_col(degp_block):
    ones = jnp.ones((NW, 1), dtype=jnp.float32)
    deg = lax.dot_general(degp_block, ones, _DEG_DOT,
                          preferred_element_type=jnp.float32)
    return lax.rsqrt(deg + 1.0)


def _dense1_body(x_ref, w_ref, degp_ref, hs_ref):
    dinv = _dinv_col(degp_ref[...])
    h = jnp.dot(x_ref[...], w_ref[...], preferred_element_type=jnp.float32)
    hs_ref[...] = h * dinv


def _dense1(x, W1, degp):
    return pl.pallas_call(
        _dense1_body,
        out_shape=jax.ShapeDtypeStruct((N, H1), jnp.float32),
    )(x, W1, degp)


def _dense2_body(p_ref, hs_ref, degp_ref, b_ref, w_ref, out_ref):
    a = p_ref[...]
    dinv = _dinv_col(degp_ref[...])
    t = (a[0] + a[1] + hs_ref[...]) * dinv + b_ref[...]
    t = jnp.maximum(t, 0.0)
    hh = jnp.dot(t, w_ref[...], preferred_element_type=jnp.float32)
    out_ref[...] = hh * dinv


def _dense2(P, hs1, degp, b1r, W2):
    return pl.pallas_call(
        _dense2_body,
        out_shape=jax.ShapeDtypeStruct((N, H2), jnp.float32),
    )(P, hs1, degp, b1r, W2)


def _dense3_body(q_ref, hs_ref, degp_ref, b_ref, w_ref, bfc_ref, out_ref):
    a = q_ref[...]
    dinv = _dinv_col(degp_ref[...])
    t = (a[0] + a[1] + hs_ref[...]) * dinv + b_ref[...]
    t = jnp.maximum(t, 0.0)
    logits = jnp.dot(t, w_ref[...], preferred_element_type=jnp.float32)
    logits = logits + bfc_ref[...]
    m = jnp.max(logits, axis=1, keepdims=True)
    e = jnp.exp(logits - m)
    out_ref[...] = e / jnp.sum(e, axis=1, keepdims=True)


def _dense3(Q, hs2, degp, b2r, Wfcp, bfcp):
    return pl.pallas_call(
        _dense3_body,
        out_shape=jax.ShapeDtypeStruct((N, CP), jnp.float32),
    )(Q, hs2, degp, b2r, Wfcp, bfcp)


def kernel(x, edge_index, W1, b1, W2, b2, Wfc, bfc):
    ei3 = edge_index.reshape(2, NCHUNKS, K)

    zeros_deg = jnp.zeros((N,), jnp.float32)
    degp = _deg_kernel(ei3, zeros_deg)

    hs1 = _dense1(x, W1, degp)

    zeros1 = jnp.zeros((N, H1), jnp.float32)
    P = _agg_h1(hs1, ei3, zeros1)

    hs2 = _dense2(P, hs1, degp, b1.reshape(1, H1), W2)

    zeros2 = jnp.zeros((N, H2), jnp.float32)
    Q = _agg_h2(hs2, ei3, zeros2)

    Wfcp = jnp.concatenate([Wfc, jnp.zeros((H2, CP - C), jnp.float32)], axis=1)
    bfcp = jnp.concatenate([bfc, jnp.full((CP - C,), -1e30, jnp.float32)])
    out = _dense3(Q, hs2, degp, b2.reshape(1, H2), Wfcp, bfcp.reshape(1, CP))
    return out[:, :C]


# speedup vs baseline: 60.1736x; 1.2423x over previous
import functools

import jax
import jax.numpy as jnp
from jax import lax
from jax.experimental import pallas as pl
from jax.experimental.pallas import tpu as pltpu
from jax.experimental.pallas import tpu_sc as plsc

N = 10000
E = 320000
D = 128
H1 = 32
H2 = 16
C = 7

NC = 2
NS = 16
NW = NC * NS
K = 128
NCHUNKS = E // K
CA = 80
CB = 76
SLAB = CA + 4
NROW = N // NS
CP = 8


def _sc_mesh():
    return plsc.VectorSubcoreMesh(
        core_axis_name="c", subcore_axis_name="s",
        num_cores=NC, num_subcores=NS,
    )


_SC_PARAMS = pltpu.CompilerParams(
    needs_layout_passes=False,
    use_tc_tiling_on_sc=False,
)


def _agg_split(c, s):
    is0 = c == 0
    count = jnp.where(is0, CA + 4 * (s == 0).astype(jnp.int32), CB)
    start = jnp.where(
        is0,
        jnp.where(s == 0, 0, 4 + s * CA),
        16 * CA + 4 + s * CB,
    )
    dma_start = jnp.minimum(start, NCHUNKS - SLAB)
    return count, dma_start, start - dma_start


def _deg_split(wid):
    count = 78 + (wid < 4).astype(jnp.int32)
    start = wid * 78 + jnp.minimum(wid, 4)
    dma_start = jnp.minimum(start, NCHUNKS - 79)
    return count, dma_start, start - dma_start


@functools.partial(
    pl.kernel,
    out_type=jax.ShapeDtypeStruct((NW, N), jnp.float32),
    mesh=_sc_mesh(),
    compiler_params=_SC_PARAMS,
    scratch_types=[
        pltpu.VMEM((79, K), jnp.int32),
        pltpu.VMEM((N,), jnp.float32),
    ],
)
def _deg_kernel(ei_hbm, zeros_hbm, out_hbm, dst_v, deg_v):
    c = lax.axis_index("c")
    s = lax.axis_index("s")
    wid = s * NC + c
    count, dma_start, off = _deg_split(wid)
    pltpu.sync_copy(zeros_hbm, deg_v)
    pltpu.sync_copy(ei_hbm.at[1].at[pl.ds(dma_start, 79)], dst_v)
    ones = jnp.full((16,), 1.0, dtype=jnp.float32)

    def body(j, carry):
        row = dst_v.at[off + j]
        for g in range(K // 16):
            idx = row[pl.ds(g * 16, 16)]
            plsc.addupdate_scatter(deg_v, [idx], ones)
        return carry

    lax.fori_loop(0, count, body, 0)
    pltpu.sync_copy(deg_v, out_hbm.at[wid])


def _make_agg(H):
    @functools.partial(
        pl.kernel,
        out_type=jax.ShapeDtypeStruct((NC, N, H), jnp.float32),
        mesh=_sc_mesh(),
        compiler_params=_SC_PARAMS,
        scratch_types=[
            pltpu.VMEM((SLAB, K), jnp.int32),
            pltpu.VMEM((SLAB, K), jnp.int32),
            pltpu.VMEM((K, H), jnp.float32),
            pltpu.VMEM((K, H), jnp.float32),
            pltpu.VMEM((K, H), jnp.float32),
            pltpu.VMEM((K, H), jnp.float32),
            pltpu.VMEM_SHARED((N, H), jnp.float32),
            pltpu.SemaphoreType.DMA,
            pltpu.SemaphoreType.DMA,
            pltpu.SemaphoreType.DMA,
            pltpu.SemaphoreType.DMA,
        ],
    )
    def agg(hs_hbm, ei_hbm, zeros_hbm, out_hbm, src_v, dst_v,
            rows0, rows1, rows2, rows3, acc, sem0, sem1, sem2, sem3):
        c = lax.axis_index("c")
        s = lax.axis_index("s")
        count, dma_start, off = _agg_split(c, s)
        rows = (rows0, rows1, rows2, rows3)
        sems = (sem0, sem1, sem2, sem3)

        pltpu.sync_copy(zeros_hbm.at[pl.ds(s * NROW, NROW)],
                        acc.at[pl.ds(s * NROW, NROW)])
        pltpu.sync_copy(ei_hbm.at[0].at[pl.ds(dma_start, SLAB)], src_v)
        pltpu.sync_copy(ei_hbm.at[1].at[pl.ds(dma_start, SLAB)], dst_v)
        plsc.subcore_barrier()

        for q in range(4):
            pltpu.async_copy(hs_hbm.at[src_v.at[off + q]], rows[q], sems[q])

        def body(i, carry):
            j = off + 4 * i
            for q in range(4):
                jq = j + q
                pltpu.make_async_copy(hs_hbm.at[src_v.at[jq]], rows[q],
                                      sems[q]).wait()
                pltpu.sync_copy(rows[q], acc.at[dst_v.at[jq]], add=True)
                jn = jnp.minimum(jq + 4, off + count - 1)
                pltpu.async_copy(hs_hbm.at[src_v.at[jn]], rows[q], sems[q])
            return carry

        lax.fori_loop(0, count // 4, body, 0)
        for q in range(4):
            pltpu.make_async_copy(hs_hbm.at[src_v.at[0]], rows[q],
                                  sems[q]).wait()

        plsc.subcore_barrier()
        pltpu.sync_copy(acc.at[pl.ds(s * NROW, NROW)],
                        out_hbm.at[c].at[pl.ds(s * NROW, NROW)])

    return agg


_agg_h1 = _make_agg(H1)
_agg_h2 = _make_agg(H2)


_DEG_DOT = (((0,), (0,)), ((), ()))


def _dinv_col(degp_block):
    ones = jnp.ones((NW, 1), dtype=jnp.float32)
    deg = lax.dot_general(degp_block, ones, _DEG_DOT,
                          preferred_element_type=jnp.float32)
    return lax.rsqrt(deg + 1.0)


def _dense1_body(x_ref, w_ref, degp_ref, hs_ref):
    dinv = _dinv_col(degp_ref[...])
    h = jnp.dot(x_ref[...], w_ref[...], preferred_element_type=jnp.float32)
    hs_ref[...] = h * dinv


def _dense1(x, W1, degp):
    return pl.pallas_call(
        _dense1_body,
        out_shape=jax.ShapeDtypeStruct((N, H1), jnp.float32),
    )(x, W1, degp)


def _dense2_body(p_ref, hs_ref, degp_ref, b_ref, w_ref, out_ref):
    a = p_ref[...]
    dinv = _dinv_col(degp_ref[...])
    t = (a[0] + a[1] + hs_ref[...]) * dinv + b_ref[...]
    t = jnp.maximum(t, 0.0)
    hh = jnp.dot(t, w_ref[...], preferred_element_type=jnp.float32)
    out_ref[...] = hh * dinv


def _dense2(P, hs1, degp, b1r, W2):
    return pl.pallas_call(
        _dense2_body,
        out_shape=jax.ShapeDtypeStruct((N, H2), jnp.float32),
    )(P, hs1, degp, b1r, W2)


def _dense3_body(q_ref, hs_ref, degp_ref, b_ref, w_ref, bfc_ref, out_ref):
    a = q_ref[...]
    dinv = _dinv_col(degp_ref[...])
    t = (a[0] + a[1] + hs_ref[...]) * dinv + b_ref[...]
    t = jnp.maximum(t, 0.0)
    logits = jnp.dot(t, w_ref[...], preferred_element_type=jnp.float32)
    logits = logits + bfc_ref[...]
    m = jnp.max(logits, axis=1, keepdims=True)
    e = jnp.exp(logits - m)
    out_ref[...] = e / jnp.sum(e, axis=1, keepdims=True)


def _dense3(Q, hs2, degp, b2r, Wfcp, bfcp):
    return pl.pallas_call(
        _dense3_body,
        out_shape=jax.ShapeDtypeStruct((N, CP), jnp.float32),
    )(Q, hs2, degp, b2r, Wfcp, bfcp)


def kernel(x, edge_index, W1, b1, W2, b2, Wfc, bfc):
    ei3 = edge_index.reshape(2, NCHUNKS, K)

    zeros_deg = jnp.zeros((N,), jnp.float32)
    degp = _deg_kernel(ei3, zeros_deg)

    hs1 = _dense1(x, W1, degp)

    zeros1 = jnp.zeros((N, H1), jnp.float32)
    P = _agg_h1(hs1, ei3, zeros1)

    hs2 = _dense2(P, hs1, degp, b1.reshape(1, H1), W2)

    zeros2 = jnp.zeros((N, H2), jnp.float32)
    Q = _agg_h2(hs2, ei3, zeros2)

    Wfcp = jnp.concatenate([Wfc, jnp.zeros((H2, CP - C), jnp.float32)], axis=1)
    bfcp = jnp.concatenate([bfc, jnp.full((CP - C,), -1e30, jnp.float32)])
    out = _dense3(Q, hs2, degp, b2.reshape(1, H2), Wfcp, bfcp.reshape(1, CP))
    return out[:, :C]


# speedup vs baseline: 62.4771x; 1.0383x over previous
import functools

import jax
import jax.numpy as jnp
from jax import lax
from jax.experimental import pallas as pl
from jax.experimental.pallas import tpu as pltpu
from jax.experimental.pallas import tpu_sc as plsc

N = 10000
E = 320000
D = 128
H1 = 32
H2 = 16
C = 7

NC = 2
NS = 16
NW = NC * NS
K = 128
NCHUNKS = E // K
CA = 80
CB = 76
SLAB = CA + 4
NROW = N // NS
CP = 8


def _sc_mesh():
    return plsc.VectorSubcoreMesh(
        core_axis_name="c", subcore_axis_name="s",
        num_cores=NC, num_subcores=NS,
    )


_SC_PARAMS = pltpu.CompilerParams(
    needs_layout_passes=False,
    use_tc_tiling_on_sc=False,
)


def _agg_split(c, s):
    is0 = c == 0
    count = jnp.where(is0, CA + 4 * (s == 0).astype(jnp.int32), CB)
    start = jnp.where(
        is0,
        jnp.where(s == 0, 0, 4 + s * CA),
        16 * CA + 4 + s * CB,
    )
    dma_start = jnp.minimum(start, NCHUNKS - SLAB)
    return count, dma_start, start - dma_start


def _deg_split(wid):
    count = 78 + (wid < 4).astype(jnp.int32)
    start = wid * 78 + jnp.minimum(wid, 4)
    dma_start = jnp.minimum(start, NCHUNKS - 79)
    return count, dma_start, start - dma_start


@functools.partial(
    pl.kernel,
    out_type=jax.ShapeDtypeStruct((NW, N), jnp.float32),
    mesh=_sc_mesh(),
    compiler_params=_SC_PARAMS,
    scratch_types=[
        pltpu.VMEM((79, K), jnp.int32),
        pltpu.VMEM((N,), jnp.float32),
    ],
)
def _deg_kernel(ei_hbm, zeros_hbm, out_hbm, dst_v, deg_v):
    c = lax.axis_index("c")
    s = lax.axis_index("s")
    wid = s * NC + c
    count, dma_start, off = _deg_split(wid)
    pltpu.sync_copy(zeros_hbm, deg_v)
    pltpu.sync_copy(ei_hbm.at[1].at[pl.ds(dma_start, 79)], dst_v)
    ones = jnp.full((16,), 1.0, dtype=jnp.float32)

    def body(j, carry):
        row = dst_v.at[off + j]
        for g in range(K // 16):
            idx = row[pl.ds(g * 16, 16)]
            plsc.addupdate_scatter(deg_v, [idx], ones)
        return carry

    lax.fori_loop(0, count, body, 0)
    pltpu.sync_copy(deg_v, out_hbm.at[wid])


def _make_agg(H):
    @functools.partial(
        pl.kernel,
        out_type=jax.ShapeDtypeStruct((NC, N, H), jnp.float32),
        mesh=_sc_mesh(),
        compiler_params=_SC_PARAMS,
        scratch_types=[
            pltpu.VMEM((SLAB, K), jnp.int32),
            pltpu.VMEM((SLAB, K), jnp.int32),
            pltpu.VMEM((K, H), jnp.float32),
            pltpu.VMEM((K, H), jnp.float32),
            pltpu.VMEM((K, H), jnp.float32),
            pltpu.VMEM((K, H), jnp.float32),
            pltpu.VMEM_SHARED((N, H), jnp.float32),
            pltpu.SemaphoreType.DMA,
            pltpu.SemaphoreType.DMA,
            pltpu.SemaphoreType.DMA,
            pltpu.SemaphoreType.DMA,
        ],
    )
    def agg(hs_hbm, ei_hbm, zeros_hbm, out_hbm, src_v, dst_v,
            rows0, rows1, rows2, rows3, acc, sem0, sem1, sem2, sem3):
        c = lax.axis_index("c")
        s = lax.axis_index("s")
        count, dma_start, off = _agg_split(c, s)
        rows = (rows0, rows1, rows2, rows3)
        sems = (sem0, sem1, sem2, sem3)

        pltpu.sync_copy(zeros_hbm.at[pl.ds(s * NROW, NROW)],
                        acc.at[pl.ds(s * NROW, NROW)])
        pltpu.sync_copy(ei_hbm.at[0].at[pl.ds(dma_start, SLAB)], src_v)
        pltpu.sync_copy(ei_hbm.at[1].at[pl.ds(dma_start, SLAB)], dst_v)
        plsc.subcore_barrier()

        for q in range(4):
            pltpu.async_copy(hs_hbm.at[src_v.at[off + q]], rows[q], sems[q])

        def body(i, carry):
            j = off + 4 * i
            for q in range(4):
                jq = j + q
                pltpu.make_async_copy(hs_hbm.at[src_v.at[jq]], rows[q],
                                      sems[q]).wait()
                pltpu.sync_copy(rows[q], acc.at[dst_v.at[jq]], add=True)
                jn = jnp.minimum(jq + 4, off + count - 1)
                pltpu.async_copy(hs_hbm.at[src_v.at[jn]], rows[q], sems[q])
            return carry

        lax.fori_loop(0, count // 4, body, 0)
        for q in range(4):
            pltpu.make_async_copy(hs_hbm.at[src_v.at[0]], rows[q],
                                  sems[q]).wait()

        plsc.subcore_barrier()
        pltpu.sync_copy(acc.at[pl.ds(s * NROW, NROW)],
                        out_hbm.at[c].at[pl.ds(s * NROW, NROW)])

    return agg


_agg_h1 = _make_agg(H1)
_agg_h2 = _make_agg(H2)


_DEG_DOT = (((0,), (0,)), ((), ()))


def _dinv_col(degp_block):
    ones = jnp.ones((NW, 1), dtype=jnp.float32)
    deg = lax.dot_general(degp_block, ones, _DEG_DOT,
                          preferred_element_type=jnp.float32)
    return lax.rsqrt(deg + 1.0)


def _dense1_body(x_ref, w_ref, degp_ref, hs_ref):
    dinv = _dinv_col(degp_ref[...])
    h = jnp.dot(x_ref[...], w_ref[...], preferred_element_type=jnp.float32)
    hs_ref[...] = h * dinv


def _dense1(x, W1, degp):
    return pl.pallas_call(
        _dense1_body,
        compiler_params=pltpu.CompilerParams(
            allow_input_fusion=[True] * 3),
        out_shape=jax.ShapeDtypeStruct((N, H1), jnp.float32),
    )(x, W1, degp)


def _dense2_body(p_ref, hs_ref, degp_ref, b_ref, w_ref, out_ref):
    a = p_ref[...]
    dinv = _dinv_col(degp_ref[...])
    t = (a[0] + a[1] + hs_ref[...]) * dinv + b_ref[...]
    t = jnp.maximum(t, 0.0)
    hh = jnp.dot(t, w_ref[...], preferred_element_type=jnp.float32)
    out_ref[...] = hh * dinv


def _dense2(P, hs1, degp, b1r, W2):
    return pl.pallas_call(
        _dense2_body,
        compiler_params=pltpu.CompilerParams(
            allow_input_fusion=[True] * 5),
        out_shape=jax.ShapeDtypeStruct((N, H2), jnp.float32),
    )(P, hs1, degp, b1r, W2)


def _dense3_body(q_ref, hs_ref, degp_ref, b_ref, w_ref, bfc_ref, out_ref):
    a = q_ref[...]
    dinv = _dinv_col(degp_ref[...])
    t = (a[0] + a[1] + hs_ref[...]) * dinv + b_ref[...]
    t = jnp.maximum(t, 0.0)
    logits = jnp.dot(t, w_ref[...], preferred_element_type=jnp.float32)
    logits = logits + bfc_ref[...]
    m = jnp.max(logits, axis=1, keepdims=True)
    e = jnp.exp(logits - m)
    out_ref[...] = e / jnp.sum(e, axis=1, keepdims=True)


def _dense3(Q, hs2, degp, b2r, Wfcp, bfcp):
    return pl.pallas_call(
        _dense3_body,
        compiler_params=pltpu.CompilerParams(
            allow_input_fusion=[True] * 6),
        out_shape=jax.ShapeDtypeStruct((N, CP), jnp.float32),
    )(Q, hs2, degp, b2r, Wfcp, bfcp)


def kernel(x, edge_index, W1, b1, W2, b2, Wfc, bfc):
    ei3 = edge_index.reshape(2, NCHUNKS, K)

    zeros_deg = jnp.zeros((N,), jnp.float32)
    degp = _deg_kernel(ei3, zeros_deg)

    hs1 = _dense1(x, W1, degp)

    zeros1 = jnp.zeros((N, H1), jnp.float32)
    P = _agg_h1(hs1, ei3, zeros1)

    hs2 = _dense2(P, hs1, degp, b1.reshape(1, H1), W2)

    zeros2 = jnp.zeros((N, H2), jnp.float32)
    Q = _agg_h2(hs2, ei3, zeros2)

    Wfcp = jnp.concatenate([Wfc, jnp.zeros((H2, CP - C), jnp.float32)], axis=1)
    bfcp = jnp.concatenate([bfc, jnp.full((CP - C,), -1e30, jnp.float32)])
    out = _dense3(Q, hs2, degp, b2.reshape(1, H2), Wfcp, bfcp.reshape(1, CP))
    return out[:, :C]
